# baseline (speedup 1.0000x reference)
import functools

import jax
import jax.numpy as jnp
from jax import lax
from jax.experimental import pallas as pl
from jax.experimental.pallas import tpu as pltpu
from jax.experimental.pallas import tpu_sc as plsc

D = 128
T = 8192
NC = 2
NS = 16
L = 16
NW = NC * NS
CPW = T // NW
SUB = 128
NSUB = CPW // SUB
GRP = 16


def _sc_body(obs_h, actp_h, rew_h, done_h, t0_h, actT_h, timeT_h, w_h, b_h,
             x_h, tout_h,
             done_v, tidx_v, aidx_v, rew_v, w_v, b_v, t0_v, tout_v,
             obs0_v, obs1_v, trow0_v, trow1_v, arow0_v, arow1_v,
             sem_o0, sem_o1, sem_t0, sem_t1, sem_a0, sem_a1, sem_x0, sem_x1):
    c = lax.axis_index("c")
    s = lax.axis_index("s")
    wid = s * NC + c
    base = wid * CPW

    pltpu.sync_copy(done_h, done_v)
    pltpu.sync_copy(w_h, w_v)
    pltpu.sync_copy(b_h, b_v)
    pltpu.sync_copy(t0_h, t0_v)
    t0 = t0_v[...]
    lanes = lax.broadcasted_iota(jnp.int32, (L,), 0)
    last = jnp.full((L,), L - 1, jnp.int32)

    def _pref_body(k, acc):
        off = k * L
        dv = done_v[pl.ds(off, L)]
        gidx = lanes + off
        vals = jnp.where(gidx < base, (gidx + t0) * dv, 0)
        return jnp.maximum(acc, vals)

    mv = plsc.parallel_loop(
        0, T // L, unroll=8, carry=jnp.zeros((L,), jnp.int32))(_pref_body)

    m_splat = plsc.cummax(mv).at[last].get(mode="promise_in_bounds")

    for kk in range(CPW // L):
        off = base + kk * L
        dv = done_v[pl.ds(off, L)]
        iv = lanes + off + t0
        eff = jnp.maximum(plsc.cummax(iv * dv), m_splat)
        tidx_v[kk // (SUB // L), pl.ds((kk % (SUB // L)) * L, L)] = iv - eff
        m_splat = eff.at[last].get(mode="promise_in_bounds")

    for sub in range(NSUB):
        sb = base + sub * SUB
        pltpu.sync_copy(actp_h.at[pl.ds(sb, SUB)], aidx_v.at[sub])
        pltpu.sync_copy(rew_h.at[pl.ds(sb, SUB)], rew_v.at[sub])
    cp_o0 = pltpu.async_copy(obs_h.at[pl.ds(base, SUB)], obs0_v, sem_o0)
    cp_t0 = pltpu.async_copy(timeT_h.at[tidx_v.at[0]], trow0_v, sem_t0)
    cp_a0 = pltpu.async_copy(actT_h.at[aidx_v.at[0]], arow0_v, sem_a0)
    cp_o1 = pltpu.async_copy(obs_h.at[pl.ds(base + SUB, SUB)], obs1_v, sem_o1)
    cp_t1 = pltpu.async_copy(timeT_h.at[tidx_v.at[1]], trow1_v, sem_t1)
    cp_a1 = pltpu.async_copy(actT_h.at[aidx_v.at[1]], arow1_v, sem_a1)

    def compute(sub, obs_v, trow_v, arow_v):
        @plsc.parallel_loop(0, SUB, GRP)
        def _(g):
            rv = rew_v[sub, pl.ds(g, GRP)]
            for rl in range(GRP):
                r = g + rl
                rs = rv.at[jnp.full((L,), rl, jnp.int32)].get(
                    mode="promise_in_bounds")
                for j in range(D // L):
                    sl = pl.ds(j * L, L)
                    obs_v[r, sl] = (obs_v[r, sl] + trow_v[r, sl]
                                    + arow_v[r, sl] + rs * w_v[sl] + b_v[sl])

    cp_o0.wait()
    cp_t0.wait()
    cp_a0.wait()
    compute(0, obs0_v, trow0_v, arow0_v)
    cp_x0 = pltpu.async_copy(obs0_v, x_h.at[pl.ds(base, SUB)], sem_x0)

    cp_o1.wait()
    cp_t1.wait()
    cp_a1.wait()
    compute(1, obs1_v, trow1_v, arow1_v)
    cp_x1 = pltpu.async_copy(obs1_v, x_h.at[pl.ds(base + SUB, SUB)], sem_x1)

    @pl.when(wid == NW - 1)
    def _():
        tout_v[...] = t0 + (jnp.int32(T) - m_splat)
        pltpu.sync_copy(tout_v, tout_h)

    cp_x0.wait()
    cp_x1.wait()


@jax.jit
def _run(obs, act_i, rew_p, done_i, t0_vec, act_table, time_table, w, b):
    mesh = plsc.VectorSubcoreMesh(core_axis_name="c", subcore_axis_name="s")
    f = functools.partial(
        pl.kernel,
        out_type=[
            jax.ShapeDtypeStruct((T, D), jnp.float32),
            jax.ShapeDtypeStruct((L,), jnp.int32),
        ],
        mesh=mesh,
        compiler_params=pltpu.CompilerParams(needs_layout_passes=False),
        scratch_types=[
            pltpu.VMEM((T,), jnp.int32),
            pltpu.VMEM((NSUB, SUB), jnp.int32),
            pltpu.VMEM((NSUB, SUB), jnp.int32),
            pltpu.VMEM((NSUB, SUB), jnp.float32),
            pltpu.VMEM((D,), jnp.float32),
            pltpu.VMEM((D,), jnp.float32),
            pltpu.VMEM((L,), jnp.int32),
            pltpu.VMEM((L,), jnp.int32),
            pltpu.VMEM((SUB, D), jnp.float32),
            pltpu.VMEM((SUB, D), jnp.float32),
            pltpu.VMEM((SUB, D), jnp.float32),
            pltpu.VMEM((SUB, D), jnp.float32),
            pltpu.VMEM((SUB, D), jnp.float32),
            pltpu.VMEM((SUB, D), jnp.float32),
            pltpu.SemaphoreType.DMA,
            pltpu.SemaphoreType.DMA,
            pltpu.SemaphoreType.DMA,
            pltpu.SemaphoreType.DMA,
            pltpu.SemaphoreType.DMA,
            pltpu.SemaphoreType.DMA,
            pltpu.SemaphoreType.DMA,
            pltpu.SemaphoreType.DMA,
        ],
    )(_sc_body)
    return f(obs, act_i, rew_p, done_i, t0_vec, act_table, time_table, w, b)


def kernel(obs, act_p, rew_p, done, time_init, act_table, time_table, rew_W, rew_b):
    act_i = act_p.astype(jnp.int32)
    done_i = done.astype(jnp.int32)
    t0_vec = jnp.full((L,), time_init.astype(jnp.int32), jnp.int32)
    x, tv = _run(obs, act_i, rew_p, done_i, t0_vec, act_table, time_table,
                 rew_W.reshape(D), rew_b)
    return (x, done, tv[0])


# speedup vs baseline: 3.2670x; 3.2670x over previous
import functools

import jax
import jax.numpy as jnp
from jax import lax
from jax.experimental import pallas as pl
from jax.experimental.pallas import tpu as pltpu
from jax.experimental.pallas import tpu_sc as plsc

D = 128
T = 8192
N_ACT = 18
NC = 2
NS = 16
L = 16
NW = NC * NS
CPW = T // NW
SUB = 128
NSUB = CPW // SUB
GRP = 16
BL = CPW + 8


def _sc_body(obs_h, actp_h, rew_h, done_h, t0_h, actT_h, timeT_h, w_h, b_h,
             x_h, tout_h,
             done_v, tidx_v, aidx_v, rew_v, w_v, b_v, t0_v, tout_v,
             obs0_v, obs1_v, ts_v, act_v,
             sem_o0, sem_o1, sem_ta, sem_tb, sem_x0, sem_x1):
    c = lax.axis_index("c")
    s = lax.axis_index("s")
    wid = s * NC + c
    base = wid * CPW

    pltpu.sync_copy(done_h, done_v)
    pltpu.sync_copy(w_h, w_v)
    pltpu.sync_copy(b_h, b_v)
    pltpu.sync_copy(t0_h, t0_v)
    pltpu.sync_copy(actT_h, act_v)
    t0 = t0_v[...]
    lanes = lax.broadcasted_iota(jnp.int32, (L,), 0)

    def _pref_body(k, acc):
        off = k * L
        dv = done_v[pl.ds(off, L)]
        gidx = lanes + off
        vals = jnp.where(gidx < base, (gidx + t0) * dv, 0)
        return jnp.maximum(acc, vals)

    mv = plsc.parallel_loop(
        0, T // L, unroll=8, carry=jnp.zeros((L,), jnp.int32))(_pref_body)
    m_s = jnp.max(mv)
    t0_s = jnp.max(t0)

    fB = pl.multiple_of(jnp.clip(((t0_s + base - m_s) >> 3) << 3, 0, T - BL), 8)

    cp_ta = pltpu.async_copy(timeT_h.at[pl.ds(0, CPW)],
                             ts_v.at[pl.ds(0, CPW)], sem_ta)
    cp_tb = pltpu.async_copy(timeT_h.at[pl.ds(fB, BL)],
                             ts_v.at[pl.ds(CPW, BL)], sem_tb)
    cp_o0 = pltpu.async_copy(obs_h.at[pl.ds(base, SUB)], obs0_v, sem_o0)
    cp_o1 = pltpu.async_copy(obs_h.at[pl.ds(base + SUB, SUB)], obs1_v, sem_o1)

    last = jnp.full((L,), L - 1, jnp.int32)
    m_splat = jnp.full((L,), m_s, jnp.int32)
    for kk in range(CPW // L):
        off = base + kk * L
        dv = done_v[pl.ds(off, L)]
        iv = lanes + off + t0
        eff = jnp.maximum(plsc.cummax(iv * dv), m_splat)
        tidx_v[kk // (SUB // L), pl.ds((kk % (SUB // L)) * L, L)] = iv - eff
        m_splat = eff.at[last].get(mode="promise_in_bounds")

    for sub in range(NSUB):
        sb = base + sub * SUB
        pltpu.sync_copy(actp_h.at[pl.ds(sb, SUB)], aidx_v.at[sub])
        pltpu.sync_copy(rew_h.at[pl.ds(sb, SUB)], rew_v.at[sub])

    fB_spl = jnp.full((L,), fB, jnp.int32)
    jis = [lanes + j * L for j in range(D // L)]

    def compute(sub, obs_v):
        def _grp(g):
            rv = rew_v[sub, pl.ds(g, GRP)]
            tv = tidx_v[sub, pl.ds(g, GRP)]
            av = aidx_v[sub, pl.ds(g, GRP)]
            lv = jnp.where(tv >= fB_spl, tv - fB_spl + CPW, tv)
            for rl in range(GRP):
                r = g + rl
                sel = jnp.full((L,), rl, jnp.int32)
                rs = rv.at[sel].get(mode="promise_in_bounds")
                l_spl = lv.at[sel].get(mode="promise_in_bounds")
                a_spl = av.at[sel].get(mode="promise_in_bounds")
                for j in range(D // L):
                    sl = pl.ds(j * L, L)
                    trow = plsc.load_gather(ts_v, [l_spl, jis[j]])
                    arow = plsc.load_gather(act_v, [a_spl, jis[j]])
                    obs_v[r, sl] = (obs_v[r, sl] + trow + arow
                                    + rs * w_v[sl] + b_v[sl])
        plsc.parallel_loop(0, SUB, GRP)(_grp)

    cp_ta.wait()
    cp_tb.wait()
    cp_o0.wait()
    compute(0, obs0_v)
    cp_x0 = pltpu.async_copy(obs0_v, x_h.at[pl.ds(base, SUB)], sem_x0)

    cp_o1.wait()
    compute(1, obs1_v)
    cp_x1 = pltpu.async_copy(obs1_v, x_h.at[pl.ds(base + SUB, SUB)], sem_x1)

    @pl.when(wid == NW - 1)
    def _():
        tout_v[...] = t0 + (jnp.int32(T) - m_splat)
        pltpu.sync_copy(tout_v, tout_h)

    cp_x0.wait()
    cp_x1.wait()


@jax.jit
def _run(obs, act_i, rew_p, done_i, t0_vec, act_table, time_table, w, b):
    mesh = plsc.VectorSubcoreMesh(core_axis_name="c", subcore_axis_name="s")
    f = functools.partial(
        pl.kernel,
        out_type=[
            jax.ShapeDtypeStruct((T, D), jnp.float32),
            jax.ShapeDtypeStruct((L,), jnp.int32),
        ],
        mesh=mesh,
        compiler_params=pltpu.CompilerParams(needs_layout_passes=False),
        scratch_types=[
            pltpu.VMEM((T,), jnp.int32),
            pltpu.VMEM((NSUB, SUB), jnp.int32),
            pltpu.VMEM((NSUB, SUB), jnp.int32),
            pltpu.VMEM((NSUB, SUB), jnp.float32),
            pltpu.VMEM((D,), jnp.float32),
            pltpu.VMEM((D,), jnp.float32),
            pltpu.VMEM((L,), jnp.int32),
            pltpu.VMEM((L,), jnp.int32),
            pltpu.VMEM((SUB, D), jnp.float32),
            pltpu.VMEM((SUB, D), jnp.float32),
            pltpu.VMEM((CPW + BL, D), jnp.float32),
            pltpu.VMEM((N_ACT, D), jnp.float32),
            pltpu.SemaphoreType.DMA,
            pltpu.SemaphoreType.DMA,
            pltpu.SemaphoreType.DMA,
            pltpu.SemaphoreType.DMA,
            pltpu.SemaphoreType.DMA,
            pltpu.SemaphoreType.DMA,
        ],
    )(_sc_body)
    return f(obs, act_i, rew_p, done_i, t0_vec, act_table, time_table, w, b)


def kernel(obs, act_p, rew_p, done, time_init, act_table, time_table, rew_W, rew_b):
    act_i = act_p.astype(jnp.int32)
    done_i = done.astype(jnp.int32)
    t0_vec = jnp.full((L,), time_init.astype(jnp.int32), jnp.int32)
    x, tv = _run(obs, act_i, rew_p, done_i, t0_vec, act_table, time_table,
                 rew_W.reshape(D), rew_b)
    return (x, done, tv[0])


# speedup vs baseline: 3.4291x; 1.0496x over previous
import functools

import jax
import jax.numpy as jnp
from jax import lax
from jax.experimental import pallas as pl
from jax.experimental.pallas import tpu as pltpu
from jax.experimental.pallas import tpu_sc as plsc

D = 128
T = 8192
N_ACT = 18
NC = 2
NS = 16
L = 16
NW = NC * NS
CPW = T // NW
SUB = 128
NSUB = CPW // SUB
GRP = 16
BL = CPW + 8


def _sc_body(obs_h, actp_h, rew_h, done_h, t0_h, actT_h, timeT_h, w_h, b_h,
             x_h, tout_h,
             done_v, tidx_v, aidx_v, rew_v, w_v, b_v, t0_v, tout_v,
             obs0_v, obs1_v, ts_v, act_v,
             sem_o0, sem_o1, sem_ta, sem_tb, sem_x0, sem_x1,
             sem_done, sem_small, sem_idx):
    c = lax.axis_index("c")
    s = lax.axis_index("s")
    wid = s * NC + c
    base = wid * CPW

    cp_done = pltpu.async_copy(done_h, done_v, sem_done)
    cp_t0 = pltpu.async_copy(t0_h, t0_v, sem_small)
    cp_w = pltpu.async_copy(w_h, w_v, sem_small)
    cp_b = pltpu.async_copy(b_h, b_v, sem_small)
    cp_act = pltpu.async_copy(actT_h, act_v, sem_small)
    cp_idx = []
    for sub in range(NSUB):
        sb = base + sub * SUB
        cp_idx.append(
            pltpu.async_copy(actp_h.at[pl.ds(sb, SUB)], aidx_v.at[sub], sem_idx))
        cp_idx.append(
            pltpu.async_copy(rew_h.at[pl.ds(sb, SUB)], rew_v.at[sub], sem_idx))
    cp_o0 = pltpu.async_copy(obs_h.at[pl.ds(base, SUB)], obs0_v, sem_o0)
    cp_o1 = pltpu.async_copy(obs_h.at[pl.ds(base + SUB, SUB)], obs1_v, sem_o1)

    cp_done.wait()
    cp_t0.wait()
    t0 = t0_v[...]
    lanes = lax.broadcasted_iota(jnp.int32, (L,), 0)

    def _pref_body(k, acc):
        off = k * L
        dv = done_v[pl.ds(off, L)]
        gidx = lanes + off
        vals = jnp.where(gidx < base, (gidx + t0) * dv, 0)
        return jnp.maximum(acc, vals)

    mv = plsc.parallel_loop(
        0, T // L, unroll=8, carry=jnp.zeros((L,), jnp.int32))(_pref_body)
    m_s = jnp.max(mv)
    t0_s = jnp.max(t0)

    fB = pl.multiple_of(jnp.clip(((t0_s + base - m_s) >> 3) << 3, 0, T - BL), 8)

    cp_ta = pltpu.async_copy(timeT_h.at[pl.ds(0, CPW)],
                             ts_v.at[pl.ds(0, CPW)], sem_ta)
    cp_tb = pltpu.async_copy(timeT_h.at[pl.ds(fB, BL)],
                             ts_v.at[pl.ds(CPW, BL)], sem_tb)

    last = jnp.full((L,), L - 1, jnp.int32)
    m_splat = jnp.full((L,), m_s, jnp.int32)
    for kk in range(CPW // L):
        off = base + kk * L
        dv = done_v[pl.ds(off, L)]
        iv = lanes + off + t0
        eff = jnp.maximum(plsc.cummax(iv * dv), m_splat)
        tidx_v[kk // (SUB // L), pl.ds((kk % (SUB // L)) * L, L)] = iv - eff
        m_splat = eff.at[last].get(mode="promise_in_bounds")

    cp_w.wait()
    cp_b.wait()
    cp_act.wait()
    for cp in cp_idx:
        cp.wait()

    fB_spl = jnp.full((L,), fB, jnp.int32)
    jis = [lanes + j * L for j in range(D // L)]
    wjs = [w_v[pl.ds(j * L, L)] for j in range(D // L)]
    bjs = [b_v[pl.ds(j * L, L)] for j in range(D // L)]

    def compute(sub, obs_v):
        def _grp(g):
            rv = rew_v[sub, pl.ds(g, GRP)]
            tv = tidx_v[sub, pl.ds(g, GRP)]
            av = aidx_v[sub, pl.ds(g, GRP)]
            lv = jnp.where(tv >= fB_spl, tv - fB_spl + CPW, tv)
            for rl in range(GRP):
                r = g + rl
                sel = jnp.full((L,), rl, jnp.int32)
                rs = rv.at[sel].get(mode="promise_in_bounds")
                l_spl = lv.at[sel].get(mode="promise_in_bounds")
                a_spl = av.at[sel].get(mode="promise_in_bounds")
                for j in range(D // L):
                    sl = pl.ds(j * L, L)
                    trow = plsc.load_gather(ts_v, [l_spl, jis[j]])
                    arow = plsc.load_gather(act_v, [a_spl, jis[j]])
                    obs_v[r, sl] = (obs_v[r, sl] + trow + arow
                                    + rs * wjs[j] + bjs[j])
        plsc.parallel_loop(0, SUB, GRP)(_grp)

    cp_ta.wait()
    cp_tb.wait()
    cp_o0.wait()
    compute(0, obs0_v)
    cp_x0 = pltpu.async_copy(obs0_v, x_h.at[pl.ds(base, SUB)], sem_x0)

    cp_o1.wait()
    compute(1, obs1_v)
    cp_x1 = pltpu.async_copy(obs1_v, x_h.at[pl.ds(base + SUB, SUB)], sem_x1)

    @pl.when(wid == NW - 1)
    def _():
        tout_v[...] = t0 + (jnp.int32(T) - m_splat)
        pltpu.sync_copy(tout_v, tout_h)

    cp_x0.wait()
    cp_x1.wait()


@jax.jit
def _run(obs, act_i, rew_p, done_i, t0_vec, act_table, time_table, w, b):
    mesh = plsc.VectorSubcoreMesh(core_axis_name="c", subcore_axis_name="s")
    f = functools.partial(
        pl.kernel,
        out_type=[
            jax.ShapeDtypeStruct((T, D), jnp.float32),
            jax.ShapeDtypeStruct((L,), jnp.int32),
        ],
        mesh=mesh,
        compiler_params=pltpu.CompilerParams(needs_layout_passes=False),
        scratch_types=[
            pltpu.VMEM((T,), jnp.int32),
            pltpu.VMEM((NSUB, SUB), jnp.int32),
            pltpu.VMEM((NSUB, SUB), jnp.int32),
            pltpu.VMEM((NSUB, SUB), jnp.float32),
            pltpu.VMEM((D,), jnp.float32),
            pltpu.VMEM((D,), jnp.float32),
            pltpu.VMEM((L,), jnp.int32),
            pltpu.VMEM((L,), jnp.int32),
            pltpu.VMEM((SUB, D), jnp.float32),
            pltpu.VMEM((SUB, D), jnp.float32),
            pltpu.VMEM((CPW + BL, D), jnp.float32),
            pltpu.VMEM((N_ACT, D), jnp.float32),
            pltpu.SemaphoreType.DMA,
            pltpu.SemaphoreType.DMA,
            pltpu.SemaphoreType.DMA,
            pltpu.SemaphoreType.DMA,
            pltpu.SemaphoreType.DMA,
            pltpu.SemaphoreType.DMA,
            pltpu.SemaphoreType.DMA,
            pltpu.SemaphoreType.DMA,
            pltpu.SemaphoreType.DMA,
        ],
    )(_sc_body)
    return f(obs, act_i, rew_p, done_i, t0_vec, act_table, time_table, w, b)


def kernel(obs, act_p, rew_p, done, time_init, act_table, time_table, rew_W, rew_b):
    act_i = act_p.astype(jnp.int32)
    done_i = done.astype(jnp.int32)
    t0_vec = jnp.full((L,), time_init.astype(jnp.int32), jnp.int32)
    x, tv = _run(obs, act_i, rew_p, done_i, t0_vec, act_table, time_table,
                 rew_W.reshape(D), rew_b)
    return (x, done, tv[0])
